# baseline (speedup 1.0000x reference)
import functools

import jax
import jax.numpy as jnp
from jax import lax
from jax.experimental import pallas as pl
from jax.experimental.pallas import tpu as pltpu
from jax.experimental.pallas import tpu_sc as plsc

BATCH = 16384
HIST = 200
N = BATCH * HIST
VOCAB_N = 1_000_000
NUM_WORKERS = 32
BPW = N // NUM_WORKERS
CHUNK = 12_800
NCHUNK = BPW // CHUNK
SEG = 10_400
NROUND = 6
TAIL_OFF = NROUND * 16 * SEG
TAIL_SEG = 200


def _make_kernel():
    mesh = plsc.VectorSubcoreMesh(core_axis_name="c", subcore_axis_name="s")

    @functools.partial(
        pl.kernel,
        mesh=mesh,
        out_type=jax.ShapeDtypeStruct((N,), jnp.float32),
        scratch_types=[
            pltpu.VMEM_SHARED((VOCAB_N,), jnp.float32),
            pltpu.VMEM((CHUNK,), jnp.int32),
            pltpu.VMEM((CHUNK,), jnp.int32),
            pltpu.VMEM((CHUNK,), jnp.float32),
            pltpu.VMEM((CHUNK,), jnp.float32),
            pltpu.SemaphoreType.DMA,
            pltpu.SemaphoreType.DMA,
            pltpu.SemaphoreType.DMA,
            pltpu.SemaphoreType.DMA,
            pltpu.SemaphoreType.DMA,
            pltpu.SemaphoreType.DMA,
        ],
    )
    def gather_kernel(tok_hbm, tab_hbm, out_hbm, tab_sp, idx0, idx1,
                      val0, val1, si0, si1, so0, so1, sga, sgb):
        s = lax.axis_index("s")
        wid = s * 2 + lax.axis_index("c")
        base = wid * BPW
        idx = (idx0, idx1)
        val = (val0, val1)
        sem_i = (si0, si1)
        sem_o = (so0, so1)
        sem_g = (sga, sgb)

        pltpu.async_copy(tok_hbm.at[pl.ds(base, CHUNK)], idx0, si0)

        def slices(r):
            if r < NROUND:
                off, ln = (r * 16 + s) * SEG, SEG
            else:
                off, ln = TAIL_OFF + s * TAIL_SEG, TAIL_SEG
            return (tab_hbm.at[pl.ds(off, ln)],
                    val[r % 2].at[pl.ds(0, ln)],
                    tab_sp.at[pl.ds(off, ln)],
                    sem_g[r % 2])

        def hop1(r):
            src, buf, _, sem = slices(r)
            pltpu.async_copy(src, buf, sem)

        def hop1_wait(r):
            src, buf, _, sem = slices(r)
            pltpu.make_async_copy(src, buf, sem).wait()

        def hop2(r):
            _, buf, dst, sem = slices(r)
            pltpu.async_copy(buf, dst, sem)

        def hop2_wait(r):
            _, buf, dst, sem = slices(r)
            pltpu.make_async_copy(buf, dst, sem).wait()

        hop1(0)
        for r in range(NROUND):
            hop1_wait(r)
            hop2(r)
            nxt = r + 1
            if nxt < NROUND:
                if nxt >= 2:
                    hop2_wait(nxt - 2)
                hop1(nxt)
            else:
                @pl.when(s < 8)
                def _():
                    hop2_wait(nxt - 2)
                    hop1(nxt)
        @pl.when(s < 8)
        def _():
            hop1_wait(NROUND)
            hop2(NROUND)
            hop2_wait(NROUND - 1)
            hop2_wait(NROUND)

        @pl.when(s >= 8)
        def _():
            hop2_wait(NROUND - 2)
            hop2_wait(NROUND - 1)

        plsc.subcore_barrier()

        for i in range(NCHUNK):
            b = i % 2
            if i >= 2:
                pltpu.make_async_copy(
                    val[b], out_hbm.at[pl.ds(base + (i - 2) * CHUNK, CHUNK)],
                    sem_o[b]).wait()
            pltpu.make_async_copy(
                tok_hbm.at[pl.ds(base + i * CHUNK, CHUNK)], idx[b],
                sem_i[b]).wait()
            pltpu.async_copy(tab_sp.at[idx[b]], val[b], sem_g[b])
            if i >= 1:
                pltpu.make_async_copy(tab_sp.at[idx[1 - b]], val[1 - b],
                                      sem_g[1 - b]).wait()
                pltpu.async_copy(
                    val[1 - b],
                    out_hbm.at[pl.ds(base + (i - 1) * CHUNK, CHUNK)],
                    sem_o[1 - b])
            if i + 1 < NCHUNK:
                pltpu.async_copy(
                    tok_hbm.at[pl.ds(base + (i + 1) * CHUNK, CHUNK)],
                    idx[1 - b], sem_i[1 - b])
        bl = (NCHUNK - 1) % 2
        pltpu.make_async_copy(tab_sp.at[idx[bl]], val[bl],
                              sem_g[bl]).wait()
        pltpu.async_copy(
            val[bl], out_hbm.at[pl.ds(base + (NCHUNK - 1) * CHUNK, CHUNK)],
            sem_o[bl])
        for i in range(NCHUNK - 2, NCHUNK):
            b = i % 2
            pltpu.make_async_copy(
                val[b], out_hbm.at[pl.ds(base + i * CHUNK, CHUNK)],
                sem_o[b]).wait()

    return gather_kernel


_GATHER = _make_kernel()


def kernel(tokens, vocab_table):
    flat = tokens.reshape(N)
    out = _GATHER(flat, vocab_table)
    return out.reshape(BATCH, HIST)


# speedup vs baseline: 1.0059x; 1.0059x over previous
import functools

import jax
import jax.numpy as jnp
from jax import lax
from jax.experimental import pallas as pl
from jax.experimental.pallas import tpu as pltpu
from jax.experimental.pallas import tpu_sc as plsc

BATCH = 16384
HIST = 200
N = BATCH * HIST
VOCAB_N = 1_000_000
NUM_WORKERS = 32
BPW = N // NUM_WORKERS
CHUNK = 12_800
NCHUNK = BPW // CHUNK
SEG = 12_480
NROUND = 5
TAIL_OFF = NROUND * 16 * SEG
TAIL_SEG = 200


def _make_kernel():
    mesh = plsc.VectorSubcoreMesh(core_axis_name="c", subcore_axis_name="s")

    @functools.partial(
        pl.kernel,
        mesh=mesh,
        out_type=jax.ShapeDtypeStruct((N,), jnp.float32),
        scratch_types=[
            pltpu.VMEM_SHARED((VOCAB_N,), jnp.float32),
            pltpu.VMEM((CHUNK,), jnp.int32),
            pltpu.VMEM((CHUNK,), jnp.int32),
            pltpu.VMEM((CHUNK,), jnp.float32),
            pltpu.VMEM((CHUNK,), jnp.float32),
            pltpu.SemaphoreType.DMA,
            pltpu.SemaphoreType.DMA,
            pltpu.SemaphoreType.DMA,
            pltpu.SemaphoreType.DMA,
            pltpu.SemaphoreType.DMA,
            pltpu.SemaphoreType.DMA,
        ],
    )
    def gather_kernel(tok_hbm, tab_hbm, out_hbm, tab_sp, idx0, idx1,
                      val0, val1, si0, si1, so0, so1, sga, sgb):
        s = lax.axis_index("s")
        wid = s * 2 + lax.axis_index("c")
        base = wid * BPW
        idx = (idx0, idx1)
        val = (val0, val1)
        sem_i = (si0, si1)
        sem_o = (so0, so1)
        sem_g = (sga, sgb)

        pltpu.async_copy(tok_hbm.at[pl.ds(base, CHUNK)], idx0, si0)

        def slices(r):
            if r < NROUND:
                off, ln = (r * 16 + s) * SEG, SEG
            else:
                off, ln = TAIL_OFF + s * TAIL_SEG, TAIL_SEG
            return (tab_hbm.at[pl.ds(off, ln)],
                    val[r % 2].at[pl.ds(0, ln)],
                    tab_sp.at[pl.ds(off, ln)],
                    sem_g[r % 2])

        def hop1(r):
            src, buf, _, sem = slices(r)
            pltpu.async_copy(src, buf, sem)

        def hop1_wait(r):
            src, buf, _, sem = slices(r)
            pltpu.make_async_copy(src, buf, sem).wait()

        def hop2(r):
            _, buf, dst, sem = slices(r)
            pltpu.async_copy(buf, dst, sem)

        def hop2_wait(r):
            _, buf, dst, sem = slices(r)
            pltpu.make_async_copy(buf, dst, sem).wait()

        hop1(0)
        for r in range(NROUND):
            hop1_wait(r)
            hop2(r)
            nxt = r + 1
            if nxt < NROUND:
                if nxt >= 2:
                    hop2_wait(nxt - 2)
                hop1(nxt)
            else:
                @pl.when(s < 8)
                def _():
                    hop2_wait(nxt - 2)
                    hop1(nxt)
        @pl.when(s < 8)
        def _():
            hop1_wait(NROUND)
            hop2(NROUND)
            hop2_wait(NROUND - 1)
            hop2_wait(NROUND)

        @pl.when(s >= 8)
        def _():
            hop2_wait(NROUND - 2)
            hop2_wait(NROUND - 1)

        plsc.subcore_barrier()

        for i in range(NCHUNK):
            b = i % 2
            if i >= 2:
                pltpu.make_async_copy(
                    val[b], out_hbm.at[pl.ds(base + (i - 2) * CHUNK, CHUNK)],
                    sem_o[b]).wait()
            pltpu.make_async_copy(
                tok_hbm.at[pl.ds(base + i * CHUNK, CHUNK)], idx[b],
                sem_i[b]).wait()
            pltpu.async_copy(tab_sp.at[idx[b]], val[b], sem_g[b])
            if i >= 1:
                pltpu.make_async_copy(tab_sp.at[idx[1 - b]], val[1 - b],
                                      sem_g[1 - b]).wait()
                pltpu.async_copy(
                    val[1 - b],
                    out_hbm.at[pl.ds(base + (i - 1) * CHUNK, CHUNK)],
                    sem_o[1 - b])
            if i + 1 < NCHUNK:
                pltpu.async_copy(
                    tok_hbm.at[pl.ds(base + (i + 1) * CHUNK, CHUNK)],
                    idx[1 - b], sem_i[1 - b])
        bl = (NCHUNK - 1) % 2
        pltpu.make_async_copy(tab_sp.at[idx[bl]], val[bl],
                              sem_g[bl]).wait()
        pltpu.async_copy(
            val[bl], out_hbm.at[pl.ds(base + (NCHUNK - 1) * CHUNK, CHUNK)],
            sem_o[bl])
        for i in range(NCHUNK - 2, NCHUNK):
            b = i % 2
            pltpu.make_async_copy(
                val[b], out_hbm.at[pl.ds(base + i * CHUNK, CHUNK)],
                sem_o[b]).wait()

    return gather_kernel


_GATHER = _make_kernel()


def kernel(tokens, vocab_table):
    flat = tokens.reshape(N)
    out = _GATHER(flat, vocab_table)
    return out.reshape(BATCH, HIST)
